# baseline (speedup 1.0000x reference)
import jax
import jax.numpy as jnp
from jax.experimental import pallas as pl
from jax.experimental.pallas import tpu as pltpu

_TINY = 1.1754943508222875e-38


def _make_kernel(V, C):
    def _rs_kernel(ids_smem, t_ref, d_ref, g_ref, idsv_ref, unifv_ref,
                   bonusv_ref, out_ref, bestv_ref, besti_ref, selt_ref,
                   seld_ref):
        j = pl.program_id(1)
        nv = pl.num_programs(1)
        rb, k, _ = d_ref.shape

        @pl.when(j == 0)
        def _init():
            bestv_ref[...] = jnp.full((rb, k), -jnp.inf, jnp.float32)
            besti_ref[...] = jnp.zeros((rb, k), jnp.int32)
            selt_ref[...] = jnp.zeros((rb, k), jnp.float32)
            seld_ref[...] = jnp.zeros((rb, k), jnp.float32)

        base = j * C
        lane = jax.lax.broadcasted_iota(jnp.int32, (rb, C), 1)
        valid = (base + lane) < V
        for kk in range(k):
            t = t_ref[:, kk, :]
            d = d_ref[:, kk, :]
            g = g_ref[:, kk, :]
            score = jnp.log(jnp.maximum(t - d, _TINY)) + g
            score = jnp.where(valid, score, -jnp.inf)
            m = jnp.max(score, axis=1, keepdims=True)
            loc = jnp.min(jnp.where(score == m, lane, C), axis=1,
                          keepdims=True)
            bv = bestv_ref[:, kk : kk + 1]
            upd = m > bv
            bestv_ref[:, kk : kk + 1] = jnp.where(upd, m, bv)
            besti_ref[:, kk : kk + 1] = jnp.where(
                upd, base + loc, besti_ref[:, kk : kk + 1])

        lane128 = jax.lax.broadcasted_iota(jnp.int32, (1, 128), 1)
        for r in range(rb):
            for kk in range(k):
                tid_s = ids_smem[0, r, kk]

                @pl.when((tid_s >= base) & (tid_s < base + C))
                def _gather(r=r, kk=kk, tid_s=tid_s):
                    off = tid_s - base
                    grp = pl.multiple_of((off // 128) * 128, 128)
                    tv = t_ref[r, kk : kk + 1, pl.ds(grp, 128)]
                    dv = d_ref[r, kk : kk + 1, pl.ds(grp, 128)]
                    msk = lane128 == (off - grp)
                    selt_ref[r : r + 1, kk : kk + 1] = jnp.sum(
                        jnp.where(msk, tv, 0.0), axis=1, keepdims=True)
                    seld_ref[r : r + 1, kk : kk + 1] = jnp.sum(
                        jnp.where(msk, dv, 0.0), axis=1, keepdims=True)

        @pl.when(j == nv - 1)
        def _epilogue():
            sel_t = selt_ref[...]
            sel_d = seld_ref[...]
            ratio = jnp.minimum(sel_t / sel_d, 1.0)
            accepted = jnp.where(unifv_ref[0] < ratio, 1, 0).astype(jnp.int32)
            kidx = jax.lax.broadcasted_iota(jnp.int32, (rb, k), 1)
            limits = jnp.min(jnp.where(accepted == 0, kidx, k), axis=1,
                             keepdims=True)
            acc_mask = kidx < limits
            after = kidx == limits
            tid = idsv_ref[0]
            out_k = jnp.where(acc_mask, tid, -1)
            bonus_col = jnp.where(out_k[:, k - 1 : k] != -1, bonusv_ref[0], -1)
            out_k = jnp.where(after, besti_ref[...], out_k)
            out_ref[0, :, :k] = out_k
            out_ref[0, :, k:] = bonus_col

    return _rs_kernel


@jax.jit
def kernel(target_with_bonus_probs, bonus_token_ids, draft_probs,
           draft_token_ids, uniform_rand, gumbel_noise):
    B, K, V = draft_probs.shape
    RB = 8
    C = 8192
    G = B // RB
    NV = (V + C - 1) // C
    ids3 = draft_token_ids.reshape(G, RB, K)
    unif3 = uniform_rand.reshape(G, RB, K)
    bonus3 = bonus_token_ids.reshape(G, RB, 1)
    out = pl.pallas_call(
        _make_kernel(V, C),
        grid=(G, NV),
        in_specs=[
            pl.BlockSpec((1, RB, K), lambda i, j: (i, 0, 0),
                         memory_space=pltpu.SMEM),
            pl.BlockSpec((RB, K + 1, C), lambda i, j: (i, 0, j)),
            pl.BlockSpec((RB, K, C), lambda i, j: (i, 0, j)),
            pl.BlockSpec((RB, K, C), lambda i, j: (i, 0, j)),
            pl.BlockSpec((1, RB, K), lambda i, j: (i, 0, 0)),
            pl.BlockSpec((1, RB, K), lambda i, j: (i, 0, 0)),
            pl.BlockSpec((1, RB, 1), lambda i, j: (i, 0, 0)),
        ],
        out_specs=pl.BlockSpec((1, RB, K + 1), lambda i, j: (i, 0, 0)),
        out_shape=jax.ShapeDtypeStruct((G, RB, K + 1), jnp.int32),
        scratch_shapes=[
            pltpu.VMEM((RB, K), jnp.float32),
            pltpu.VMEM((RB, K), jnp.int32),
            pltpu.VMEM((RB, K), jnp.float32),
            pltpu.VMEM((RB, K), jnp.float32),
        ],
        compiler_params=pltpu.CompilerParams(
            dimension_semantics=("parallel", "arbitrary"),
        ),
    )(ids3, target_with_bonus_probs, draft_probs, gumbel_noise, ids3, unif3,
      bonus3)
    return out.reshape(B, K + 1)
